# baseline (speedup 1.0000x reference)
import functools

import jax
import jax.numpy as jnp
from jax import lax
from jax.experimental import pallas as pl
from jax.experimental.pallas import tpu as pltpu
from jax.experimental.pallas import tpu_sc as plsc

N = 1000000
D = 64
CC = 4
R = 256
NBUF = 4
LOOKAHEAD = 2


def _make_kernel():
    info = plsc.get_sparse_core_info()
    nc, ns = info.num_cores, info.num_subcores
    nw = nc * ns
    nchunk = (N + R - 1) // R
    nit = (nchunk + nw - 1) // nw
    nit = ((nit + NBUF - 1) // NBUF) * NBUF
    nouter = nit // NBUF
    last_base = N - R

    mesh = plsc.VectorSubcoreMesh(core_axis_name="c", subcore_axis_name="s")

    scratch = (
        [pltpu.VMEM((8 * D,), jnp.float32)]
        + [pltpu.VMEM((R * D,), jnp.float32) for _ in range(NBUF)]
        + [pltpu.VMEM((R * CC,), jnp.int32) for _ in range(NBUF)]
        + [pltpu.SemaphoreType.DMA for _ in range(3 * NBUF)]
    )

    @functools.partial(
        pl.kernel,
        out_type=jax.ShapeDtypeStruct((N * D,), jnp.float32),
        mesh=mesh,
        scratch_types=scratch,
        compiler_params=pltpu.CompilerParams(needs_layout_passes=False),
    )
    def sc_kernel(x_hbm, c_hbm, t_hbm, o_hbm, tbl_v, *bufs):
        xbufs = bufs[0:NBUF]
        cbufs = bufs[NBUF:2 * NBUF]
        xsems = bufs[2 * NBUF:3 * NBUF]
        csems = bufs[3 * NBUF:4 * NBUF]
        osems = bufs[4 * NBUF:5 * NBUF]

        w = lax.axis_index("s") * nc + lax.axis_index("c")

        def base_of(j):
            q = w + nw * j
            return jnp.minimum(q * R, last_base)

        def issue_in(j, b):
            base = base_of(j)
            pltpu.make_async_copy(
                x_hbm.at[pl.ds(base * D, R * D)], xbufs[b], xsems[b]).start()
            pltpu.make_async_copy(
                c_hbm.at[pl.ds(base * CC, R * CC)], cbufs[b], csems[b]).start()

        def wait_in(b):
            pltpu.make_async_copy(
                x_hbm.at[pl.ds(0, R * D)], xbufs[b], xsems[b]).wait()
            pltpu.make_async_copy(
                c_hbm.at[pl.ds(0, R * CC)], cbufs[b], csems[b]).wait()

        def issue_out(j, b):
            base = base_of(j)
            pltpu.make_async_copy(
                xbufs[b], o_hbm.at[pl.ds(base * D, R * D)], osems[b]).start()

        def wait_out(b):
            pltpu.make_async_copy(
                xbufs[b], o_hbm.at[pl.ds(0, R * D)], osems[b]).wait()

        def compute(b):
            xb, cb = xbufs[b], cbufs[b]
            iota4 = lax.iota(jnp.int32, 16) * CC

            def group_body(g, carry):
                gb = g * 16
                cbase = gb * CC + iota4
                c1 = plsc.load_gather(cb, [cbase + 1])
                c2 = plsc.load_gather(cb, [cbase + 2])
                c3 = plsc.load_gather(cb, [cbase + 3])
                idx16 = (c1 & 1) + 2 * (c2 & 1) + 4 * (c3 & 1)
                toff16 = idx16 * D
                for r2 in range(16):
                    row = gb + r2
                    toff = toff16[r2]
                    for cblk in range(D // 16):
                        tv = tbl_v[pl.ds(toff + cblk * 16, 16)]
                        plsc.addupdate(
                            xb.at[pl.ds(row * D + cblk * 16, 16)], tv)
                return carry

            lax.fori_loop(0, R // 16, group_body, 0)

        pltpu.sync_copy(t_hbm, tbl_v)

        for b in range(LOOKAHEAD):
            issue_in(b, b)

        def outer(k, carry):
            for b in range(NBUF):
                j = NBUF * k + b
                bnext = (b + LOOKAHEAD) % NBUF
                if b < LOOKAHEAD:
                    @pl.when(k >= 1)
                    def _():
                        wait_out(bnext)
                    issue_in(j + LOOKAHEAD, bnext)
                else:
                    wait_out(bnext)

                    @pl.when(k < nouter - 1)
                    def _():
                        issue_in(j + LOOKAHEAD, bnext)
                wait_in(b)
                compute(b)
                issue_out(j, b)
            return carry

        lax.fori_loop(0, nouter, outer, 0)

        for b in range(NBUF - LOOKAHEAD, NBUF):
            wait_out(b)

    return sc_kernel


def kernel(x_up_F, x_up_C, target_res_embedding):
    out = _make_kernel()(
        x_up_F.reshape(-1),
        x_up_C.reshape(-1),
        target_res_embedding.reshape(-1),
    )
    return out.reshape(N, D)
